# Initial kernel scaffold; baseline (speedup 1.0000x reference)
import jax
import jax.numpy as jnp
from jax.experimental import pallas as pl


def kernel(x, paths):
    raise NotImplementedError("write your pallas kernel here")


# speedup vs baseline: 5.9716x; 5.9716x over previous
import functools

import jax
import jax.numpy as jnp
from jax import lax
from jax.experimental import pallas as pl
from jax.experimental.pallas import tpu as pltpu
from jax.experimental.pallas import tpu_sc as plsc

BS, C, H, W = 2, 96, 224, 224
NP = 4
HW = H * W
ROWS = BS * C
NWORKERS = 32
ROWS_PER_W = ROWS // NWORKERS
NCHUNK = 4
K = HW // NCHUNK
VECS = K // 16


def _sc_body(x_hbm, h_hbm, w_hbm, out_hbm, table_v, h_v, w_v, o_v):
    nc = 2
    wid = lax.axis_index("s") * nc + lax.axis_index("c")

    def row_body(j, carry):
        t = wid * ROWS_PER_W + j
        pltpu.sync_copy(x_hbm.at[t], table_v)
        b = t // C
        c = t - b * C

        def path_body(p, carry):
            r = b * (NP * C) + p * C + c

            def chunk_body(m, carry):
                off = m * K
                pltpu.sync_copy(h_hbm.at[p, pl.ds(off, K)], h_v)
                pltpu.sync_copy(w_hbm.at[p, pl.ds(off, K)], w_v)

                def vec_body(v, carry):
                    s = v * 16
                    hv = h_v[pl.ds(s, 16)]
                    wv = w_v[pl.ds(s, 16)]
                    iv = hv * W + wv
                    o_v[pl.ds(s, 16)] = plsc.load_gather(table_v, [iv])
                    return carry

                lax.fori_loop(0, VECS, vec_body, 0)
                pltpu.sync_copy(o_v, out_hbm.at[r, pl.ds(off, K)])
                return carry

            return lax.fori_loop(0, NCHUNK, chunk_body, carry)

        return lax.fori_loop(0, NP, path_body, carry)

    lax.fori_loop(0, ROWS_PER_W, row_body, 0)


def kernel(x, paths):
    bs, idim, h, w = x.shape
    x_flat = x.reshape(bs * idim, h * w)
    h_in = paths[:, :, 0]
    w_in = paths[:, :, 1]
    mesh = plsc.VectorSubcoreMesh(core_axis_name="c", subcore_axis_name="s")
    run = pl.kernel(
        _sc_body,
        out_type=jax.ShapeDtypeStruct((ROWS * NP, HW), jnp.float32),
        mesh=mesh,
        compiler_params=pltpu.CompilerParams(needs_layout_passes=False),
        scratch_types=[
            pltpu.VMEM((HW,), jnp.float32),
            pltpu.VMEM((K,), jnp.int32),
            pltpu.VMEM((K,), jnp.int32),
            pltpu.VMEM((K,), jnp.float32),
        ],
    )
    out = run(x_flat, h_in, w_in)
    return out.reshape(bs, NP * idim, HW)


# speedup vs baseline: 7.6846x; 1.2869x over previous
import functools

import jax
import jax.numpy as jnp
from jax import lax
from jax.experimental import pallas as pl
from jax.experimental.pallas import tpu as pltpu
from jax.experimental.pallas import tpu_sc as plsc

BS, C, H, W = 2, 96, 224, 224
NP = 4
HW = H * W
ROWS = BS * C
NWORKERS = 32
ROWS_PER_W = ROWS // NWORKERS
NCHUNK = 4
K = HW // NCHUNK
VECS = K // 16


def _sc_body(x_hbm, h_hbm, w_hbm, out_hbm, table_v, h_v, w_v, o_v):
    nc = 2
    wid = lax.axis_index("s") * nc + lax.axis_index("c")

    def row_body(j, carry):
        t = wid * ROWS_PER_W + j
        pltpu.sync_copy(x_hbm.at[t], table_v)
        b = t // C
        c = t - b * C

        def path_body(p, carry):
            r = b * (NP * C) + p * C + c

            def chunk_body(m, carry):
                off = m * K
                pltpu.sync_copy(h_hbm.at[p, pl.ds(off, K)], h_v)
                pltpu.sync_copy(w_hbm.at[p, pl.ds(off, K)], w_v)

                @plsc.parallel_loop(0, K, step=16, unroll=8)
                def _gather(s):
                    hv = h_v[pl.ds(s, 16)]
                    wv = w_v[pl.ds(s, 16)]
                    iv = hv * W + wv
                    o_v[pl.ds(s, 16)] = plsc.load_gather(table_v, [iv])
                pltpu.sync_copy(o_v, out_hbm.at[r, pl.ds(off, K)])
                return carry

            return lax.fori_loop(0, NCHUNK, chunk_body, carry)

        return lax.fori_loop(0, NP, path_body, carry)

    lax.fori_loop(0, ROWS_PER_W, row_body, 0)


def kernel(x, paths):
    bs, idim, h, w = x.shape
    x_flat = x.reshape(bs * idim, h * w)
    h_in = paths[:, :, 0]
    w_in = paths[:, :, 1]
    mesh = plsc.VectorSubcoreMesh(core_axis_name="c", subcore_axis_name="s")
    run = pl.kernel(
        _sc_body,
        out_type=jax.ShapeDtypeStruct((ROWS * NP, HW), jnp.float32),
        mesh=mesh,
        compiler_params=pltpu.CompilerParams(needs_layout_passes=False),
        scratch_types=[
            pltpu.VMEM((HW,), jnp.float32),
            pltpu.VMEM((K,), jnp.int32),
            pltpu.VMEM((K,), jnp.int32),
            pltpu.VMEM((K,), jnp.float32),
        ],
    )
    out = run(x_flat, h_in, w_in)
    return out.reshape(bs, NP * idim, HW)


# speedup vs baseline: 10.5042x; 1.3669x over previous
import functools

import jax
import jax.numpy as jnp
from jax import lax
from jax.experimental import pallas as pl
from jax.experimental.pallas import tpu as pltpu
from jax.experimental.pallas import tpu_sc as plsc

BS, C, H, W = 2, 96, 224, 224
NP = 4
HW = H * W
ROWS = BS * C
NWORKERS = 32
ROWS_PER_W = ROWS // NWORKERS
NPAIR = ROWS_PER_W // 2
NCHUNK = 14
K = HW // NCHUNK
NSUB = (NP * HW) // K


def _sc_body(x_hbm, h_hbm, w_hbm, out_hbm,
             t0_v, t1_v, idx_v, o0_v, o1_v, idx_sh):
    nc = 2
    cid = lax.axis_index("c")
    sid = lax.axis_index("s")
    wid = sid * nc + cid

    for j in range((NSUB + 15) // 16):
        cix = sid + 16 * j

        @pl.when(cix < NSUB)
        def _pre():
            flat0 = cix * K
            p = flat0 // HW
            base = flat0 - p * HW
            pltpu.sync_copy(h_hbm.at[p, pl.ds(base, K)], idx_v)
            pltpu.sync_copy(w_hbm.at[p, pl.ds(base, K)], o0_v)

            @plsc.parallel_loop(0, K, step=16, unroll=8)
            def _flat(s):
                o1_v[pl.ds(s, 16)] = (idx_v[pl.ds(s, 16)] * W
                                      + o0_v[pl.ds(s, 16)])

            pltpu.sync_copy(o1_v, idx_sh.at[pl.ds(flat0, K)])

    plsc.subcore_barrier()

    def pair_body(q, carry):
        t0 = wid * ROWS_PER_W + 2 * q
        t1 = t0 + 1
        pltpu.sync_copy(x_hbm.at[t0], t0_v)
        pltpu.sync_copy(x_hbm.at[t1], t1_v)
        b = t0 // C
        c = t0 - b * C

        def path_body(p, carry):
            r0 = b * (NP * C) + p * C + c
            r1 = r0 + 1

            def chunk_body(m, carry):
                off = m * K
                pltpu.sync_copy(idx_sh.at[pl.ds(p * HW + off, K)], idx_v)

                @plsc.parallel_loop(0, K, step=16, unroll=8)
                def _gather(s):
                    iv = idx_v[pl.ds(s, 16)]
                    o0_v[pl.ds(s, 16)] = plsc.load_gather(t0_v, [iv])
                    o1_v[pl.ds(s, 16)] = plsc.load_gather(t1_v, [iv])

                pltpu.sync_copy(o0_v, out_hbm.at[r0, pl.ds(off, K)])
                pltpu.sync_copy(o1_v, out_hbm.at[r1, pl.ds(off, K)])
                return carry

            return lax.fori_loop(0, NCHUNK, chunk_body, carry)

        return lax.fori_loop(0, NP, path_body, carry)

    lax.fori_loop(0, NPAIR, pair_body, 0)


def kernel(x, paths):
    bs, idim, h, w = x.shape
    x_flat = lax.bitcast_convert_type(x.reshape(bs * idim, h * w), jnp.int32)
    h_in = paths[:, :, 0]
    w_in = paths[:, :, 1]
    mesh = plsc.VectorSubcoreMesh(core_axis_name="c", subcore_axis_name="s")
    run = pl.kernel(
        _sc_body,
        out_type=jax.ShapeDtypeStruct((ROWS * NP, HW), jnp.int32),
        mesh=mesh,
        compiler_params=pltpu.CompilerParams(needs_layout_passes=False),
        scratch_types=[
            pltpu.VMEM((HW,), jnp.int32),
            pltpu.VMEM((HW,), jnp.int32),
            pltpu.VMEM((K,), jnp.int32),
            pltpu.VMEM((K,), jnp.int32),
            pltpu.VMEM((K,), jnp.int32),
            pltpu.VMEM_SHARED((NP * HW,), jnp.int32),
        ],
    )
    out = run(x_flat, h_in, w_in)
    return lax.bitcast_convert_type(out, jnp.float32).reshape(
        bs, NP * idim, HW)


# speedup vs baseline: 13.6739x; 1.3018x over previous
import functools

import jax
import jax.numpy as jnp
from jax import lax
from jax.experimental import pallas as pl
from jax.experimental.pallas import tpu as pltpu
from jax.experimental.pallas import tpu_sc as plsc

BS, C, H, W = 2, 96, 224, 224
NP = 4
HW = H * W
ROWS = BS * C
NWORKERS = 32
ROWS_PER_W = ROWS // NWORKERS
NPAIR = ROWS_PER_W // 2
K = 1792
UCH = (NP * HW) // K
PCH = HW // K
NSUB = (NP * HW) // K


def _sc_body(x_hbm, h_hbm, w_hbm, out_hbm,
             t0_v, t1_v, idx_va, idx_vb, oa0_v, oa1_v, ob0_v, ob1_v,
             sem_ia, sem_ib, sem_oa, sem_ob, idx_sh):
    nc = 2
    cid = lax.axis_index("c")
    sid = lax.axis_index("s")
    wid = sid * nc + cid
    idx_bufs = (idx_va, idx_vb)
    out_bufs = ((oa0_v, oa1_v), (ob0_v, ob1_v))
    isems = (sem_ia, sem_ib)
    osems = (sem_oa, sem_ob)

    for j in range((NSUB + 15) // 16):
        cix = sid + 16 * j

        @pl.when(cix < NSUB)
        def _pre():
            flat0 = cix * K
            p = flat0 // HW
            base = flat0 - p * HW
            pltpu.sync_copy(h_hbm.at[p, pl.ds(base, K)], idx_va)
            pltpu.sync_copy(w_hbm.at[p, pl.ds(base, K)], oa0_v)

            @plsc.parallel_loop(0, K, step=16, unroll=8)
            def _flat(s):
                oa1_v[pl.ds(s, 16)] = (idx_va[pl.ds(s, 16)] * W
                                       + oa0_v[pl.ds(s, 16)])

            pltpu.sync_copy(oa1_v, idx_sh.at[pl.ds(flat0, K)])

    plsc.subcore_barrier()

    def pair_body(q, carry):
        t0 = wid * ROWS_PER_W + 2 * q
        t1 = t0 + 1
        pltpu.sync_copy(x_hbm.at[t0], t0_v)
        pltpu.sync_copy(x_hbm.at[t1], t1_v)
        b = t0 // C
        c = t0 - b * C
        r_base = b * (NP * C) + c

        pltpu.async_copy(idx_sh.at[pl.ds(0, K)], idx_va, sem_ia)
        pltpu.async_copy(idx_sh.at[pl.ds(K, K)], idx_vb, sem_ib)

        def u_pair(uu, carry):
            for ph in range(2):
                u = uu * 2 + ph
                iv_ref = idx_bufs[ph]
                ob0, ob1 = out_bufs[ph]
                p = u // PCH
                m = u - p * PCH
                off = m * K
                r0 = r_base + p * C

                pltpu.make_async_copy(
                    idx_sh.at[pl.ds(0, K)], iv_ref, isems[ph]).wait()
                @pl.when(u >= 2)
                def _drain():
                    pltpu.make_async_copy(
                        ob0, out_hbm.at[r0, pl.ds(0, K)], osems[ph]).wait()
                    pltpu.make_async_copy(
                        ob1, out_hbm.at[r0, pl.ds(0, K)], osems[ph]).wait()

                @plsc.parallel_loop(0, K, step=16, unroll=8)
                def _gather(s):
                    iv = iv_ref[pl.ds(s, 16)]
                    ob0[pl.ds(s, 16)] = plsc.load_gather(t0_v, [iv])
                    ob1[pl.ds(s, 16)] = plsc.load_gather(t1_v, [iv])

                pltpu.async_copy(ob0, out_hbm.at[r0, pl.ds(off, K)],
                                 osems[ph])
                pltpu.async_copy(ob1, out_hbm.at[r0 + 1, pl.ds(off, K)],
                                 osems[ph])

                @pl.when(u + 2 < UCH)
                def _prefetch():
                    pltpu.async_copy(
                        idx_sh.at[pl.ds((u + 2) * K, K)], iv_ref, isems[ph])
            return carry

        lax.fori_loop(0, UCH // 2, u_pair, carry)

        for ph in range(2):
            ob0, ob1 = out_bufs[ph]
            pltpu.make_async_copy(
                ob0, out_hbm.at[0, pl.ds(0, K)], osems[ph]).wait()
            pltpu.make_async_copy(
                ob1, out_hbm.at[0, pl.ds(0, K)], osems[ph]).wait()
        return carry

    lax.fori_loop(0, NPAIR, pair_body, 0)


def kernel(x, paths):
    bs, idim, h, w = x.shape
    x_flat = lax.bitcast_convert_type(x.reshape(bs * idim, h * w), jnp.int32)
    h_in = paths[:, :, 0]
    w_in = paths[:, :, 1]
    mesh = plsc.VectorSubcoreMesh(core_axis_name="c", subcore_axis_name="s")
    run = pl.kernel(
        _sc_body,
        out_type=jax.ShapeDtypeStruct((ROWS * NP, HW), jnp.int32),
        mesh=mesh,
        compiler_params=pltpu.CompilerParams(needs_layout_passes=False),
        scratch_types=[
            pltpu.VMEM((HW,), jnp.int32),
            pltpu.VMEM((HW,), jnp.int32),
            pltpu.VMEM((K,), jnp.int32),
            pltpu.VMEM((K,), jnp.int32),
            pltpu.VMEM((K,), jnp.int32),
            pltpu.VMEM((K,), jnp.int32),
            pltpu.VMEM((K,), jnp.int32),
            pltpu.VMEM((K,), jnp.int32),
            pltpu.SemaphoreType.DMA,
            pltpu.SemaphoreType.DMA,
            pltpu.SemaphoreType.DMA,
            pltpu.SemaphoreType.DMA,
            pltpu.VMEM_SHARED((NP * HW,), jnp.int32),
        ],
    )
    out = run(x_flat, h_in, w_in)
    return lax.bitcast_convert_type(out, jnp.float32).reshape(
        bs, NP * idim, HW)


# speedup vs baseline: 14.3773x; 1.0514x over previous
import functools

import jax
import jax.numpy as jnp
from jax import lax
from jax.experimental import pallas as pl
from jax.experimental.pallas import tpu as pltpu
from jax.experimental.pallas import tpu_sc as plsc

BS, C, H, W = 2, 96, 224, 224
NP = 4
HW = H * W
ROWS = BS * C
NWORKERS = 32
ROWS_PER_W = ROWS // NWORKERS
NPAIR = ROWS_PER_W // 2
K = 3584
UCH = (NP * HW) // K
PCH = HW // K
NSUB = (NP * HW) // K


def _sc_body(x_hbm, h_hbm, w_hbm, out_hbm,
             t0_v, t1_v, idx_va, idx_vb, oa0_v, oa1_v, ob0_v, ob1_v,
             sem_ia, sem_ib, sem_oa, sem_ob, idx_sh):
    nc = 2
    cid = lax.axis_index("c")
    sid = lax.axis_index("s")
    wid = sid * nc + cid
    idx_bufs = (idx_va, idx_vb)
    out_bufs = ((oa0_v, oa1_v), (ob0_v, ob1_v))
    isems = (sem_ia, sem_ib)
    osems = (sem_oa, sem_ob)

    for j in range((NSUB + 15) // 16):
        cix = sid + 16 * j

        @pl.when(cix < NSUB)
        def _pre():
            flat0 = cix * K
            p = flat0 // HW
            base = flat0 - p * HW
            pltpu.sync_copy(h_hbm.at[p, pl.ds(base, K)], oa0_v)
            pltpu.sync_copy(w_hbm.at[p, pl.ds(base, K)], oa1_v)

            @plsc.parallel_loop(0, K // 2, step=16, unroll=8)
            def _flat(s):
                s2 = pl.multiple_of(2 * s, 32)
                fa = (oa0_v[pl.ds(s2, 16)] * W
                      + oa1_v[pl.ds(s2, 16)])
                fb = (oa0_v[pl.ds(s2 + 16, 16)] * W
                      + oa1_v[pl.ds(s2 + 16, 16)])
                idx_va[pl.ds(s, 16)] = fa | (fb << 16)

            pltpu.sync_copy(
                idx_va,
                idx_sh.at[pl.ds(pl.multiple_of(cix * (K // 2), K // 2),
                                K // 2)])

    plsc.subcore_barrier()

    def pair_body(q, carry):
        t0 = wid * ROWS_PER_W + 2 * q
        t1 = t0 + 1
        pltpu.sync_copy(x_hbm.at[t0], t0_v)
        pltpu.sync_copy(x_hbm.at[t1], t1_v)
        b = t0 // C
        c = t0 - b * C
        r_base = b * (NP * C) + c

        pltpu.async_copy(idx_sh.at[pl.ds(0, K // 2)], idx_va, sem_ia)
        pltpu.async_copy(idx_sh.at[pl.ds(K // 2, K // 2)], idx_vb, sem_ib)

        def u_pair(uu, carry):
            for ph in range(2):
                u = uu * 2 + ph
                iv_ref = idx_bufs[ph]
                ob0, ob1 = out_bufs[ph]
                p = u // PCH
                m = u - p * PCH
                off = m * K
                r0 = r_base + p * C

                pltpu.make_async_copy(
                    idx_sh.at[pl.ds(0, K // 2)], iv_ref, isems[ph]).wait()
                @pl.when(u >= 2)
                def _drain():
                    pltpu.make_async_copy(
                        ob0, out_hbm.at[r0, pl.ds(0, K)], osems[ph]).wait()
                    pltpu.make_async_copy(
                        ob1, out_hbm.at[r0, pl.ds(0, K)], osems[ph]).wait()

                @plsc.parallel_loop(0, K // 2, step=16, unroll=4)
                def _gather(s):
                    s2 = pl.multiple_of(2 * s, 32)
                    pk = iv_ref[pl.ds(s, 16)]
                    ia = pk & 0xFFFF
                    ib = lax.shift_right_logical(pk, 16)
                    ob0[pl.ds(s2, 16)] = plsc.load_gather(t0_v, [ia])
                    ob0[pl.ds(s2 + 16, 16)] = plsc.load_gather(t0_v, [ib])
                    ob1[pl.ds(s2, 16)] = plsc.load_gather(t1_v, [ia])
                    ob1[pl.ds(s2 + 16, 16)] = plsc.load_gather(t1_v, [ib])

                pltpu.async_copy(ob0, out_hbm.at[r0, pl.ds(off, K)],
                                 osems[ph])
                pltpu.async_copy(ob1, out_hbm.at[r0 + 1, pl.ds(off, K)],
                                 osems[ph])

                @pl.when(u + 2 < UCH)
                def _prefetch():
                    pltpu.async_copy(
                        idx_sh.at[pl.ds(
                            pl.multiple_of((u + 2) * (K // 2), K // 2),
                            K // 2)],
                        iv_ref, isems[ph])
            return carry

        lax.fori_loop(0, UCH // 2, u_pair, carry)

        for ph in range(2):
            ob0, ob1 = out_bufs[ph]
            pltpu.make_async_copy(
                ob0, out_hbm.at[0, pl.ds(0, K)], osems[ph]).wait()
            pltpu.make_async_copy(
                ob1, out_hbm.at[0, pl.ds(0, K)], osems[ph]).wait()
        return carry

    lax.fori_loop(0, NPAIR, pair_body, 0)


def kernel(x, paths):
    bs, idim, h, w = x.shape
    x_flat = lax.bitcast_convert_type(x.reshape(bs * idim, h * w), jnp.int32)
    h_in = paths[:, :, 0]
    w_in = paths[:, :, 1]
    mesh = plsc.VectorSubcoreMesh(core_axis_name="c", subcore_axis_name="s")
    run = pl.kernel(
        _sc_body,
        out_type=jax.ShapeDtypeStruct((ROWS * NP, HW), jnp.int32),
        mesh=mesh,
        compiler_params=pltpu.CompilerParams(needs_layout_passes=False),
        scratch_types=[
            pltpu.VMEM((HW,), jnp.int32),
            pltpu.VMEM((HW,), jnp.int32),
            pltpu.VMEM((K // 2,), jnp.int32),
            pltpu.VMEM((K // 2,), jnp.int32),
            pltpu.VMEM((K,), jnp.int32),
            pltpu.VMEM((K,), jnp.int32),
            pltpu.VMEM((K,), jnp.int32),
            pltpu.VMEM((K,), jnp.int32),
            pltpu.SemaphoreType.DMA,
            pltpu.SemaphoreType.DMA,
            pltpu.SemaphoreType.DMA,
            pltpu.SemaphoreType.DMA,
            pltpu.VMEM_SHARED((NP * HW // 2,), jnp.int32),
        ],
    )
    out = run(x_flat, h_in, w_in)
    return lax.bitcast_convert_type(out, jnp.float32).reshape(
        bs, NP * idim, HW)


# speedup vs baseline: 14.4135x; 1.0025x over previous
import functools

import jax
import jax.numpy as jnp
from jax import lax
from jax.experimental import pallas as pl
from jax.experimental.pallas import tpu as pltpu
from jax.experimental.pallas import tpu_sc as plsc

BS, C, H, W = 2, 96, 224, 224
NP = 4
HW = H * W
ROWS = BS * C
NWORKERS = 32
ROWS_PER_W = ROWS // NWORKERS
NPAIR = ROWS_PER_W // 2
K = 3584
UCH = (NP * HW) // K
PCH = HW // K
NSUB = (NP * HW) // K


def _sc_body(x_hbm, h_hbm, w_hbm, out_hbm,
             t0_v, t1_v, idx_va, idx_vb, oa_v, ob_v,
             sem_ia, sem_ib, sem_oa, sem_ob, idx_sh):
    nc = 2
    cid = lax.axis_index("c")
    sid = lax.axis_index("s")
    wid = sid * nc + cid
    idx_bufs = (idx_va, idx_vb)
    out_bufs = (oa_v, ob_v)
    isems = (sem_ia, sem_ib)
    osems = (sem_oa, sem_ob)

    for j in range((NSUB + 15) // 16):
        cix = sid + 16 * j

        @pl.when(cix < NSUB)
        def _pre():
            flat0 = cix * K
            p = flat0 // HW
            base = flat0 - p * HW
            pltpu.sync_copy(h_hbm.at[p, pl.ds(base, K)], oa_v.at[0])
            pltpu.sync_copy(w_hbm.at[p, pl.ds(base, K)], oa_v.at[1])

            @plsc.parallel_loop(0, K // 2, step=16, unroll=8)
            def _flat(s):
                s2 = pl.multiple_of(2 * s, 32)
                fa = (oa_v[0, pl.ds(s2, 16)] * W
                      + oa_v[1, pl.ds(s2, 16)])
                fb = (oa_v[0, pl.ds(s2 + 16, 16)] * W
                      + oa_v[1, pl.ds(s2 + 16, 16)])
                idx_va[pl.ds(s, 16)] = fa | (fb << 16)

            pltpu.sync_copy(
                idx_va,
                idx_sh.at[pl.ds(pl.multiple_of(cix * (K // 2), K // 2),
                                K // 2)])

    plsc.subcore_barrier()

    def pair_body(q, carry):
        t0 = wid * ROWS_PER_W + 2 * q
        t1 = t0 + 1
        pltpu.sync_copy(x_hbm.at[t0], t0_v)
        pltpu.sync_copy(x_hbm.at[t1], t1_v)
        b = t0 // C
        c = t0 - b * C
        r_base = b * (NP * C) + c

        pltpu.async_copy(idx_sh.at[pl.ds(0, K // 2)], idx_va, sem_ia)
        pltpu.async_copy(idx_sh.at[pl.ds(K // 2, K // 2)], idx_vb, sem_ib)

        def u_pair(uu, carry):
            for ph in range(2):
                u = uu * 2 + ph
                iv_ref = idx_bufs[ph]
                ob = out_bufs[ph]
                p = u // PCH
                m = u - p * PCH
                off = m * K
                r0 = r_base + p * C

                pltpu.make_async_copy(
                    idx_sh.at[pl.ds(0, K // 2)], iv_ref, isems[ph]).wait()
                @pl.when(u >= 2)
                def _drain():
                    pltpu.make_async_copy(
                        ob, out_hbm.at[pl.ds(r0, 2), pl.ds(0, K)],
                        osems[ph]).wait()

                @plsc.parallel_loop(0, K // 2, step=16, unroll=4)
                def _gather(s):
                    s2 = pl.multiple_of(2 * s, 32)
                    pk = iv_ref[pl.ds(s, 16)]
                    ia = pk & 0xFFFF
                    ib = lax.shift_right_logical(pk, 16)
                    ob[0, pl.ds(s2, 16)] = plsc.load_gather(t0_v, [ia])
                    ob[0, pl.ds(s2 + 16, 16)] = plsc.load_gather(t0_v, [ib])
                    ob[1, pl.ds(s2, 16)] = plsc.load_gather(t1_v, [ia])
                    ob[1, pl.ds(s2 + 16, 16)] = plsc.load_gather(t1_v, [ib])

                pltpu.async_copy(ob, out_hbm.at[pl.ds(r0, 2), pl.ds(off, K)],
                                 osems[ph])

                @pl.when(u + 2 < UCH)
                def _prefetch():
                    pltpu.async_copy(
                        idx_sh.at[pl.ds(
                            pl.multiple_of((u + 2) * (K // 2), K // 2),
                            K // 2)],
                        iv_ref, isems[ph])
            return carry

        lax.fori_loop(0, UCH // 2, u_pair, carry)

        for ph in range(2):
            pltpu.make_async_copy(
                out_bufs[ph], out_hbm.at[pl.ds(0, 2), pl.ds(0, K)],
                osems[ph]).wait()
        return carry

    lax.fori_loop(0, NPAIR, pair_body, 0)


def kernel(x, paths):
    bs, idim, h, w = x.shape
    x_flat = lax.bitcast_convert_type(x.reshape(bs * idim, h * w), jnp.int32)
    h_in = paths[:, :, 0]
    w_in = paths[:, :, 1]
    mesh = plsc.VectorSubcoreMesh(core_axis_name="c", subcore_axis_name="s")
    run = pl.kernel(
        _sc_body,
        out_type=jax.ShapeDtypeStruct((ROWS * NP, HW), jnp.int32),
        mesh=mesh,
        compiler_params=pltpu.CompilerParams(needs_layout_passes=False),
        scratch_types=[
            pltpu.VMEM((HW,), jnp.int32),
            pltpu.VMEM((HW,), jnp.int32),
            pltpu.VMEM((K // 2,), jnp.int32),
            pltpu.VMEM((K // 2,), jnp.int32),
            pltpu.VMEM((2, K), jnp.int32),
            pltpu.VMEM((2, K), jnp.int32),
            pltpu.SemaphoreType.DMA,
            pltpu.SemaphoreType.DMA,
            pltpu.SemaphoreType.DMA,
            pltpu.SemaphoreType.DMA,
            pltpu.VMEM_SHARED((NP * HW // 2,), jnp.int32),
        ],
    )
    out = run(x_flat, h_in, w_in)
    return lax.bitcast_convert_type(out, jnp.float32).reshape(
        bs, NP * idim, HW)


# speedup vs baseline: 24.6719x; 1.7117x over previous
import functools

import jax
import jax.numpy as jnp
from jax import lax
from jax.experimental import pallas as pl
from jax.experimental.pallas import tpu as pltpu
from jax.experimental.pallas import tpu_sc as plsc

BS, C, H, W = 2, 96, 224, 224
NP = 4
HW = H * W
ROWS = BS * C
NWORKERS = 32
ROWS_PER_W = ROWS // NWORKERS
NPAIR = ROWS_PER_W // 2
K = 3584
UCH = (NP * HW) // K
PCH = HW // K
KP = 1792
NSUBP = (NP * HW) // KP


def _sc_body(x_hbm, h_hbm, w_hbm, out_hbm,
             t0_v, t1_v, idx_va, idx_vb, oa_v, ob_v, h_s, w_s,
             sem_ia, sem_ib, sem_oa, sem_ob, idx_sh):
    nc = 2
    cid = lax.axis_index("c")
    sid = lax.axis_index("s")
    wid = sid * nc + cid
    idx_bufs = (idx_va, idx_vb)
    out_bufs = (oa_v, ob_v)
    isems = (sem_ia, sem_ib)
    osems = (sem_oa, sem_ob)

    for j in range(NSUBP // 16):
        cix = sid + 16 * j
        flat0 = cix * KP
        p = flat0 // HW
        base = flat0 - p * HW
        pltpu.sync_copy(h_hbm.at[p, pl.ds(base, KP)], h_s)
        pltpu.sync_copy(w_hbm.at[p, pl.ds(base, KP)], w_s)

        @plsc.parallel_loop(0, KP // 2, step=16, unroll=8)
        def _flat(s):
            s2 = pl.multiple_of(2 * s, 32)
            fa = h_s[pl.ds(s2, 16)] * W + w_s[pl.ds(s2, 16)]
            fb = h_s[pl.ds(s2 + 16, 16)] * W + w_s[pl.ds(s2 + 16, 16)]
            idx_va[pl.ds(s, 16)] = fa | (fb << 16)

        pltpu.sync_copy(
            idx_va.at[pl.ds(0, KP // 2)],
            idx_sh.at[pl.ds(pl.multiple_of(cix * (KP // 2), KP // 2),
                            KP // 2)])

    plsc.subcore_barrier()

    def pair_body(q, carry):
        t0 = wid * ROWS_PER_W + 2 * q
        t1 = t0 + 1
        pltpu.sync_copy(x_hbm.at[t0], t0_v)
        pltpu.sync_copy(x_hbm.at[t1], t1_v)
        b = t0 // C
        c = t0 - b * C
        r_base = b * (NP * C) + c

        pltpu.async_copy(idx_sh.at[pl.ds(0, K // 2)], idx_va, sem_ia)
        pltpu.async_copy(idx_sh.at[pl.ds(K // 2, K // 2)], idx_vb, sem_ib)

        def u_pair(uu, carry):
            for ph in range(2):
                u = uu * 2 + ph
                iv_ref = idx_bufs[ph]
                ob = out_bufs[ph]
                p = u // PCH
                m = u - p * PCH
                off = m * K
                r0 = r_base + p * C

                pltpu.make_async_copy(
                    idx_sh.at[pl.ds(0, K // 2)], iv_ref, isems[ph]).wait()
                @pl.when(u >= 2)
                def _drain():
                    pltpu.make_async_copy(
                        ob, out_hbm.at[pl.ds(r0, 2), pl.ds(0, K)],
                        osems[ph]).wait()

                @plsc.parallel_loop(0, K // 2, step=16, unroll=4)
                def _gather(s):
                    s2 = pl.multiple_of(2 * s, 32)
                    pk = iv_ref[pl.ds(s, 16)]
                    ia = pk & 0xFFFF
                    ib = lax.shift_right_logical(pk, 16)
                    ob[0, pl.ds(s2, 16)] = plsc.load_gather(t0_v, [ia])
                    ob[0, pl.ds(s2 + 16, 16)] = plsc.load_gather(t0_v, [ib])
                    ob[1, pl.ds(s2, 16)] = plsc.load_gather(t1_v, [ia])
                    ob[1, pl.ds(s2 + 16, 16)] = plsc.load_gather(t1_v, [ib])

                pltpu.async_copy(ob, out_hbm.at[pl.ds(r0, 2), pl.ds(off, K)],
                                 osems[ph])

                @pl.when(u + 2 < UCH)
                def _prefetch():
                    pltpu.async_copy(
                        idx_sh.at[pl.ds(
                            pl.multiple_of((u + 2) * (K // 2), K // 2),
                            K // 2)],
                        iv_ref, isems[ph])
            return carry

        lax.fori_loop(0, UCH // 2, u_pair, carry)

        for ph in range(2):
            pltpu.make_async_copy(
                out_bufs[ph], out_hbm.at[pl.ds(0, 2), pl.ds(0, K)],
                osems[ph]).wait()
        return carry

    lax.fori_loop(0, NPAIR, pair_body, 0)


def kernel(x, paths):
    bs, idim, h, w = x.shape
    x_flat = x.reshape(bs * idim, h * w)
    h_in = paths[:, :, 0]
    w_in = paths[:, :, 1]
    mesh = plsc.VectorSubcoreMesh(core_axis_name="c", subcore_axis_name="s")
    run = pl.kernel(
        _sc_body,
        out_type=jax.ShapeDtypeStruct((ROWS * NP, HW), jnp.float32),
        mesh=mesh,
        compiler_params=pltpu.CompilerParams(needs_layout_passes=False),
        scratch_types=[
            pltpu.VMEM((HW,), jnp.float32),
            pltpu.VMEM((HW,), jnp.float32),
            pltpu.VMEM((K // 2,), jnp.int32),
            pltpu.VMEM((K // 2,), jnp.int32),
            pltpu.VMEM((2, K), jnp.float32),
            pltpu.VMEM((2, K), jnp.float32),
            pltpu.VMEM((KP,), jnp.int32),
            pltpu.VMEM((KP,), jnp.int32),
            pltpu.SemaphoreType.DMA,
            pltpu.SemaphoreType.DMA,
            pltpu.SemaphoreType.DMA,
            pltpu.SemaphoreType.DMA,
            pltpu.VMEM_SHARED((NP * HW // 2,), jnp.int32),
        ],
    )
    out = run(x_flat, h_in, w_in)
    return out.reshape(bs, NP * idim, HW)


# speedup vs baseline: 25.3489x; 1.0274x over previous
import functools

import jax
import jax.numpy as jnp
from jax import lax
from jax.experimental import pallas as pl
from jax.experimental.pallas import tpu as pltpu
from jax.experimental.pallas import tpu_sc as plsc

BS, C, H, W = 2, 96, 224, 224
NP = 4
HW = H * W
ROWS = BS * C
NWORKERS = 32
ROWS_PER_W = ROWS // NWORKERS
NPAIR = ROWS_PER_W // 2
K = 3584
UCH = (NP * HW) // K
PCH = HW // K
KP = 1792
NSUBP = (NP * HW) // KP


def _sc_body(x_hbm, h_hbm, w_hbm, out_hbm,
             t0_v, t1_v, idx_va, idx_vb, oa_v, ob_v, h_s, w_s,
             sem_ia, sem_ib, sem_oa, sem_ob, sem_t0, sem_t1, idx_sh):
    nc = 2
    cid = lax.axis_index("c")
    sid = lax.axis_index("s")
    wid = sid * nc + cid
    idx_bufs = (idx_va, idx_vb)
    out_bufs = (oa_v, ob_v)
    isems = (sem_ia, sem_ib)
    osems = (sem_oa, sem_ob)

    pltpu.async_copy(x_hbm.at[wid * ROWS_PER_W], t0_v, sem_t0)
    pltpu.async_copy(x_hbm.at[wid * ROWS_PER_W + 1], t1_v, sem_t1)

    for j in range(NSUBP // 16):
        cix = sid + 16 * j
        flat0 = cix * KP
        p = flat0 // HW
        base = flat0 - p * HW
        pltpu.sync_copy(h_hbm.at[p, pl.ds(base, KP)], h_s)
        pltpu.sync_copy(w_hbm.at[p, pl.ds(base, KP)], w_s)

        @plsc.parallel_loop(0, KP // 2, step=16, unroll=8)
        def _flat(s):
            s2 = pl.multiple_of(2 * s, 32)
            fa = h_s[pl.ds(s2, 16)] * W + w_s[pl.ds(s2, 16)]
            fb = h_s[pl.ds(s2 + 16, 16)] * W + w_s[pl.ds(s2 + 16, 16)]
            idx_va[pl.ds(s, 16)] = fa | (fb << 16)

        pltpu.sync_copy(
            idx_va.at[pl.ds(0, KP // 2)],
            idx_sh.at[pl.ds(pl.multiple_of(cix * (KP // 2), KP // 2),
                            KP // 2)])

    plsc.subcore_barrier()

    def pair_body(q, carry):
        t0 = wid * ROWS_PER_W + 2 * q
        t1 = t0 + 1
        pltpu.make_async_copy(x_hbm.at[t0], t0_v, sem_t0).wait()
        pltpu.make_async_copy(x_hbm.at[t1], t1_v, sem_t1).wait()
        b = t0 // C
        c = t0 - b * C
        r_base = b * (NP * C) + c

        pltpu.async_copy(idx_sh.at[pl.ds(0, K // 2)], idx_va, sem_ia)
        pltpu.async_copy(idx_sh.at[pl.ds(K // 2, K // 2)], idx_vb, sem_ib)

        def u_pair(uu, carry):
            for ph in range(2):
                u = uu * 2 + ph
                iv_ref = idx_bufs[ph]
                ob = out_bufs[ph]
                p = u // PCH
                m = u - p * PCH
                off = m * K
                r0 = r_base + p * C

                pltpu.make_async_copy(
                    idx_sh.at[pl.ds(0, K // 2)], iv_ref, isems[ph]).wait()
                @pl.when(u >= 2)
                def _drain():
                    pltpu.make_async_copy(
                        ob, out_hbm.at[pl.ds(r0, 2), pl.ds(0, K)],
                        osems[ph]).wait()

                @plsc.parallel_loop(0, K // 2, step=16, unroll=4)
                def _gather(s):
                    s2 = pl.multiple_of(2 * s, 32)
                    pk = iv_ref[pl.ds(s, 16)]
                    ia = pk & 0xFFFF
                    ib = lax.shift_right_logical(pk, 16)
                    ob[0, pl.ds(s2, 16)] = plsc.load_gather(t0_v, [ia])
                    ob[0, pl.ds(s2 + 16, 16)] = plsc.load_gather(t0_v, [ib])
                    ob[1, pl.ds(s2, 16)] = plsc.load_gather(t1_v, [ia])
                    ob[1, pl.ds(s2 + 16, 16)] = plsc.load_gather(t1_v, [ib])

                pltpu.async_copy(ob, out_hbm.at[pl.ds(r0, 2), pl.ds(off, K)],
                                 osems[ph])

                @pl.when(u + 2 < UCH)
                def _prefetch():
                    pltpu.async_copy(
                        idx_sh.at[pl.ds(
                            pl.multiple_of((u + 2) * (K // 2), K // 2),
                            K // 2)],
                        iv_ref, isems[ph])
            return carry

        lax.fori_loop(0, UCH // 2, u_pair, carry)

        @pl.when(q + 1 < NPAIR)
        def _next_tables():
            pltpu.async_copy(x_hbm.at[t0 + 2], t0_v, sem_t0)
            pltpu.async_copy(x_hbm.at[t1 + 2], t1_v, sem_t1)

        for ph in range(2):
            pltpu.make_async_copy(
                out_bufs[ph], out_hbm.at[pl.ds(0, 2), pl.ds(0, K)],
                osems[ph]).wait()
        return carry

    lax.fori_loop(0, NPAIR, pair_body, 0)


def kernel(x, paths):
    bs, idim, h, w = x.shape
    x_flat = x.reshape(bs * idim, h * w)
    h_in = paths[:, :, 0]
    w_in = paths[:, :, 1]
    mesh = plsc.VectorSubcoreMesh(core_axis_name="c", subcore_axis_name="s")
    run = pl.kernel(
        _sc_body,
        out_type=jax.ShapeDtypeStruct((ROWS * NP, HW), jnp.float32),
        mesh=mesh,
        compiler_params=pltpu.CompilerParams(needs_layout_passes=False),
        scratch_types=[
            pltpu.VMEM((HW,), jnp.float32),
            pltpu.VMEM((HW,), jnp.float32),
            pltpu.VMEM((K // 2,), jnp.int32),
            pltpu.VMEM((K // 2,), jnp.int32),
            pltpu.VMEM((2, K), jnp.float32),
            pltpu.VMEM((2, K), jnp.float32),
            pltpu.VMEM((KP,), jnp.int32),
            pltpu.VMEM((KP,), jnp.int32),
            pltpu.SemaphoreType.DMA,
            pltpu.SemaphoreType.DMA,
            pltpu.SemaphoreType.DMA,
            pltpu.SemaphoreType.DMA,
            pltpu.SemaphoreType.DMA,
            pltpu.SemaphoreType.DMA,
            pltpu.VMEM_SHARED((NP * HW // 2,), jnp.int32),
        ],
    )
    out = run(x_flat, h_in, w_in)
    return out.reshape(bs, NP * idim, HW)


# speedup vs baseline: 26.1560x; 1.0318x over previous
import functools

import jax
import jax.numpy as jnp
from jax import lax
from jax.experimental import pallas as pl
from jax.experimental.pallas import tpu as pltpu
from jax.experimental.pallas import tpu_sc as plsc

BS, C, H, W = 2, 96, 224, 224
NP = 4
HW = H * W
ROWS = BS * C
NWORKERS = 32
ROWS_PER_W = ROWS // NWORKERS
NPAIR = ROWS_PER_W // 2
K = 3584
UCH = (NP * HW) // K
PCH = HW // K
KP = 1792
NSUBP = (NP * HW) // KP


def _sc_body(x_hbm, h_hbm, w_hbm, out_hbm,
             t0_v, t1_v, idx_va, idx_vb, oa_v, ob_v, h_s, w_s, pk_a, pk_b,
             sem_ia, sem_ib, sem_oa, sem_ob, sem_t0, sem_t1, sem_pk, idx_sh):
    nc = 2
    cid = lax.axis_index("c")
    sid = lax.axis_index("s")
    wid = sid * nc + cid
    idx_bufs = (idx_va, idx_vb)
    out_bufs = (oa_v, ob_v)
    isems = (sem_ia, sem_ib)
    osems = (sem_oa, sem_ob)

    pltpu.async_copy(x_hbm.at[wid * ROWS_PER_W], t0_v, sem_t0)
    pltpu.async_copy(x_hbm.at[wid * ROWS_PER_W + 1], t1_v, sem_t1)

    NJ = NSUBP // 16
    hw_sets = ((idx_va, idx_vb, sem_ia, sem_ib, pk_a),
               (h_s, w_s, sem_oa, sem_ob, pk_b))

    def _issue_p0(j):
        cix = sid + 16 * j
        flat0 = cix * KP
        p = flat0 // HW
        base = flat0 - p * HW
        hb, wb, sh, sw, _ = hw_sets[j % 2]
        pltpu.async_copy(h_hbm.at[p, pl.ds(base, KP)], hb, sh)
        pltpu.async_copy(w_hbm.at[p, pl.ds(base, KP)], wb, sw)

    _issue_p0(0)
    for j in range(NJ):
        hb, wb, sh, sw, pkb = hw_sets[j % 2]
        pltpu.make_async_copy(h_hbm.at[0, pl.ds(0, KP)], hb, sh).wait()
        pltpu.make_async_copy(h_hbm.at[0, pl.ds(0, KP)], wb, sw).wait()
        if j + 1 < NJ:
            _issue_p0(j + 1)
        if j >= 2:
            pltpu.make_async_copy(
                pkb, idx_sh.at[pl.ds(0, KP // 2)], sem_pk).wait()

        @plsc.parallel_loop(0, KP // 2, step=16, unroll=8)
        def _flat(s):
            s2 = pl.multiple_of(2 * s, 32)
            fa = hb[pl.ds(s2, 16)] * W + wb[pl.ds(s2, 16)]
            fb = hb[pl.ds(s2 + 16, 16)] * W + wb[pl.ds(s2 + 16, 16)]
            pkb[pl.ds(s, 16)] = fa | (fb << 16)

        cix = sid + 16 * j
        pltpu.async_copy(
            pkb,
            idx_sh.at[pl.ds(pl.multiple_of(cix * (KP // 2), KP // 2),
                            KP // 2)],
            sem_pk)

    for _ in range(2):
        pltpu.make_async_copy(
            pk_a, idx_sh.at[pl.ds(0, KP // 2)], sem_pk).wait()

    plsc.subcore_barrier()

    def pair_body(q, carry):
        t0 = wid * ROWS_PER_W + 2 * q
        t1 = t0 + 1
        pltpu.make_async_copy(x_hbm.at[t0], t0_v, sem_t0).wait()
        pltpu.make_async_copy(x_hbm.at[t1], t1_v, sem_t1).wait()
        b = t0 // C
        c = t0 - b * C
        r_base = b * (NP * C) + c

        pltpu.async_copy(idx_sh.at[pl.ds(0, K // 2)], idx_va, sem_ia)
        pltpu.async_copy(idx_sh.at[pl.ds(K // 2, K // 2)], idx_vb, sem_ib)

        def u_pair(uu, carry):
            for ph in range(2):
                u = uu * 2 + ph
                iv_ref = idx_bufs[ph]
                ob = out_bufs[ph]
                p = u // PCH
                m = u - p * PCH
                off = m * K
                r0 = r_base + p * C

                pltpu.make_async_copy(
                    idx_sh.at[pl.ds(0, K // 2)], iv_ref, isems[ph]).wait()
                @pl.when(u >= 2)
                def _drain():
                    pltpu.make_async_copy(
                        ob, out_hbm.at[pl.ds(r0, 2), pl.ds(0, K)],
                        osems[ph]).wait()

                @plsc.parallel_loop(0, K // 2, step=16, unroll=4)
                def _gather(s):
                    s2 = pl.multiple_of(2 * s, 32)
                    pk = iv_ref[pl.ds(s, 16)]
                    ia = pk & 0xFFFF
                    ib = lax.shift_right_logical(pk, 16)
                    ob[0, pl.ds(s2, 16)] = plsc.load_gather(t0_v, [ia])
                    ob[0, pl.ds(s2 + 16, 16)] = plsc.load_gather(t0_v, [ib])
                    ob[1, pl.ds(s2, 16)] = plsc.load_gather(t1_v, [ia])
                    ob[1, pl.ds(s2 + 16, 16)] = plsc.load_gather(t1_v, [ib])

                pltpu.async_copy(ob, out_hbm.at[pl.ds(r0, 2), pl.ds(off, K)],
                                 osems[ph])

                @pl.when(u + 2 < UCH)
                def _prefetch():
                    pltpu.async_copy(
                        idx_sh.at[pl.ds(
                            pl.multiple_of((u + 2) * (K // 2), K // 2),
                            K // 2)],
                        iv_ref, isems[ph])
            return carry

        lax.fori_loop(0, UCH // 2, u_pair, carry)

        @pl.when(q + 1 < NPAIR)
        def _next_tables():
            pltpu.async_copy(x_hbm.at[t0 + 2], t0_v, sem_t0)
            pltpu.async_copy(x_hbm.at[t1 + 2], t1_v, sem_t1)

        for ph in range(2):
            pltpu.make_async_copy(
                out_bufs[ph], out_hbm.at[pl.ds(0, 2), pl.ds(0, K)],
                osems[ph]).wait()
        return carry

    lax.fori_loop(0, NPAIR, pair_body, 0)


def kernel(x, paths):
    bs, idim, h, w = x.shape
    x_flat = x.reshape(bs * idim, h * w)
    h_in = paths[:, :, 0]
    w_in = paths[:, :, 1]
    mesh = plsc.VectorSubcoreMesh(core_axis_name="c", subcore_axis_name="s")
    run = pl.kernel(
        _sc_body,
        out_type=jax.ShapeDtypeStruct((ROWS * NP, HW), jnp.float32),
        mesh=mesh,
        compiler_params=pltpu.CompilerParams(needs_layout_passes=False),
        scratch_types=[
            pltpu.VMEM((HW,), jnp.float32),
            pltpu.VMEM((HW,), jnp.float32),
            pltpu.VMEM((K // 2,), jnp.int32),
            pltpu.VMEM((K // 2,), jnp.int32),
            pltpu.VMEM((2, K), jnp.float32),
            pltpu.VMEM((2, K), jnp.float32),
            pltpu.VMEM((KP,), jnp.int32),
            pltpu.VMEM((KP,), jnp.int32),
            pltpu.VMEM((KP // 2,), jnp.int32),
            pltpu.VMEM((KP // 2,), jnp.int32),
            pltpu.SemaphoreType.DMA,
            pltpu.SemaphoreType.DMA,
            pltpu.SemaphoreType.DMA,
            pltpu.SemaphoreType.DMA,
            pltpu.SemaphoreType.DMA,
            pltpu.SemaphoreType.DMA,
            pltpu.SemaphoreType.DMA,
            pltpu.VMEM_SHARED((NP * HW // 2,), jnp.int32),
        ],
    )
    out = run(x_flat, h_in, w_in)
    return out.reshape(bs, NP * idim, HW)


# speedup vs baseline: 26.4412x; 1.0109x over previous
import functools

import jax
import jax.numpy as jnp
from jax import lax
from jax.experimental import pallas as pl
from jax.experimental.pallas import tpu as pltpu
from jax.experimental.pallas import tpu_sc as plsc

BS, C, H, W = 2, 96, 224, 224
NP = 4
HW = H * W
ROWS = BS * C
NWORKERS = 32
ROWS_PER_W = ROWS // NWORKERS
NPAIR = ROWS_PER_W // 2
K = 3584
UCH = (NP * HW) // K
PCH = HW // K
KP = 1792
NSUBP = (NP * HW) // KP


def _sc_body(x_hbm, h_hbm, w_hbm, out_hbm,
             t0_v, t1_v, idx_va, idx_vb, oa_v, ob_v, h_s, w_s, pk_a, pk_b,
             sem_ia, sem_ib, sem_oa, sem_ob, sem_t0, sem_t1, sem_pk, idx_sh):
    nc = 2
    cid = lax.axis_index("c")
    sid = lax.axis_index("s")
    wid = sid * nc + cid
    idx_bufs = (idx_va, idx_vb)
    out_bufs = (oa_v, ob_v)
    isems = (sem_ia, sem_ib)
    osems = (sem_oa, sem_ob)

    pltpu.async_copy(x_hbm.at[wid * ROWS_PER_W], t0_v, sem_t0)
    pltpu.async_copy(x_hbm.at[wid * ROWS_PER_W + 1], t1_v, sem_t1)

    NJ = NSUBP // 16
    hw_sets = ((idx_va, idx_vb, sem_ia, sem_ib, pk_a),
               (h_s, w_s, sem_oa, sem_ob, pk_b))

    def _issue_p0(j):
        cix = sid + 16 * j
        flat0 = cix * KP
        p = flat0 // HW
        base = flat0 - p * HW
        hb, wb, sh, sw, _ = hw_sets[j % 2]
        pltpu.async_copy(h_hbm.at[p, pl.ds(base, KP)], hb, sh)
        pltpu.async_copy(w_hbm.at[p, pl.ds(base, KP)], wb, sw)

    _issue_p0(0)
    for j in range(NJ):
        hb, wb, sh, sw, pkb = hw_sets[j % 2]
        pltpu.make_async_copy(h_hbm.at[0, pl.ds(0, KP)], hb, sh).wait()
        pltpu.make_async_copy(h_hbm.at[0, pl.ds(0, KP)], wb, sw).wait()
        if j + 1 < NJ:
            _issue_p0(j + 1)
        if j >= 2:
            pltpu.make_async_copy(
                pkb, idx_sh.at[pl.ds(0, KP // 2)], sem_pk).wait()

        @plsc.parallel_loop(0, KP // 2, step=16, unroll=8)
        def _flat(s):
            s2 = pl.multiple_of(2 * s, 32)
            fa = hb[pl.ds(s2, 16)] * W + wb[pl.ds(s2, 16)]
            fb = hb[pl.ds(s2 + 16, 16)] * W + wb[pl.ds(s2 + 16, 16)]
            pkb[pl.ds(s, 16)] = fa | (fb << 16)

        cix = sid + 16 * j
        pltpu.async_copy(
            pkb,
            idx_sh.at[pl.ds(pl.multiple_of(cix * (KP // 2), KP // 2),
                            KP // 2)],
            sem_pk)

    for _ in range(2):
        pltpu.make_async_copy(
            pk_a, idx_sh.at[pl.ds(0, KP // 2)], sem_pk).wait()

    plsc.subcore_barrier()

    def pair_body(q, carry):
        t0 = wid * ROWS_PER_W + 2 * q
        t1 = t0 + 1
        pltpu.make_async_copy(x_hbm.at[t0], t0_v, sem_t0).wait()
        pltpu.make_async_copy(x_hbm.at[t1], t1_v, sem_t1).wait()
        b = t0 // C
        c = t0 - b * C
        r_base = b * (NP * C) + c

        pltpu.async_copy(idx_sh.at[pl.ds(0, K // 2)], idx_va, sem_ia)
        pltpu.async_copy(idx_sh.at[pl.ds(K // 2, K // 2)], idx_vb, sem_ib)

        def u_pair(uu, carry):
            for ph in range(2):
                u = uu * 2 + ph
                iv_ref = idx_bufs[ph]
                ob = out_bufs[ph]
                p = u // PCH
                m = u - p * PCH
                off = m * K
                r0 = r_base + p * C

                pltpu.make_async_copy(
                    idx_sh.at[pl.ds(0, K // 2)], iv_ref, isems[ph]).wait()
                @pl.when(u >= 2)
                def _drain():
                    pltpu.make_async_copy(
                        ob, out_hbm.at[pl.ds(r0, 2), pl.ds(0, K)],
                        osems[ph]).wait()

                @plsc.parallel_loop(0, K // 2, step=16, unroll=8)
                def _gather(s):
                    s2 = pl.multiple_of(2 * s, 32)
                    pk = iv_ref[pl.ds(s, 16)]
                    ia = pk & 0xFFFF
                    ib = lax.shift_right_logical(pk, 16)
                    ob[0, pl.ds(s2, 16)] = plsc.load_gather(t0_v, [ia])
                    ob[0, pl.ds(s2 + 16, 16)] = plsc.load_gather(t0_v, [ib])
                    ob[1, pl.ds(s2, 16)] = plsc.load_gather(t1_v, [ia])
                    ob[1, pl.ds(s2 + 16, 16)] = plsc.load_gather(t1_v, [ib])

                pltpu.async_copy(ob, out_hbm.at[pl.ds(r0, 2), pl.ds(off, K)],
                                 osems[ph])

                @pl.when(u + 2 < UCH)
                def _prefetch():
                    pltpu.async_copy(
                        idx_sh.at[pl.ds(
                            pl.multiple_of((u + 2) * (K // 2), K // 2),
                            K // 2)],
                        iv_ref, isems[ph])
            return carry

        lax.fori_loop(0, UCH // 2, u_pair, carry)

        @pl.when(q + 1 < NPAIR)
        def _next_tables():
            pltpu.async_copy(x_hbm.at[t0 + 2], t0_v, sem_t0)
            pltpu.async_copy(x_hbm.at[t1 + 2], t1_v, sem_t1)

        for ph in range(2):
            pltpu.make_async_copy(
                out_bufs[ph], out_hbm.at[pl.ds(0, 2), pl.ds(0, K)],
                osems[ph]).wait()
        return carry

    lax.fori_loop(0, NPAIR, pair_body, 0)


def kernel(x, paths):
    bs, idim, h, w = x.shape
    x_flat = x.reshape(bs * idim, h * w)
    h_in = paths[:, :, 0]
    w_in = paths[:, :, 1]
    mesh = plsc.VectorSubcoreMesh(core_axis_name="c", subcore_axis_name="s")
    run = pl.kernel(
        _sc_body,
        out_type=jax.ShapeDtypeStruct((ROWS * NP, HW), jnp.float32),
        mesh=mesh,
        compiler_params=pltpu.CompilerParams(needs_layout_passes=False),
        scratch_types=[
            pltpu.VMEM((HW,), jnp.float32),
            pltpu.VMEM((HW,), jnp.float32),
            pltpu.VMEM((K // 2,), jnp.int32),
            pltpu.VMEM((K // 2,), jnp.int32),
            pltpu.VMEM((2, K), jnp.float32),
            pltpu.VMEM((2, K), jnp.float32),
            pltpu.VMEM((KP,), jnp.int32),
            pltpu.VMEM((KP,), jnp.int32),
            pltpu.VMEM((KP // 2,), jnp.int32),
            pltpu.VMEM((KP // 2,), jnp.int32),
            pltpu.SemaphoreType.DMA,
            pltpu.SemaphoreType.DMA,
            pltpu.SemaphoreType.DMA,
            pltpu.SemaphoreType.DMA,
            pltpu.SemaphoreType.DMA,
            pltpu.SemaphoreType.DMA,
            pltpu.SemaphoreType.DMA,
            pltpu.VMEM_SHARED((NP * HW // 2,), jnp.int32),
        ],
    )
    out = run(x_flat, h_in, w_in)
    return out.reshape(bs, NP * idim, HW)
